# Initial kernel scaffold; baseline (speedup 1.0000x reference)
import jax
import jax.numpy as jnp
from jax.experimental import pallas as pl


def kernel(x, edge_index, batch, W1a, b1a, W1b, b1b, bn_gamma, bn_beta, gatW, att_src, att_dst, gat_b, fcW, fcb, h0W1, h0b1, h0W2, h0b2, h1W1, h1b1, h1W2, h1b2, h2W1, h2b1, h2W2, h2b2):
    raise NotImplementedError("write your pallas kernel here")


# speedup vs baseline: 13.1334x; 13.1334x over previous
import functools

import jax
import jax.numpy as jnp
from jax import lax
from jax.experimental import pallas as pl
from jax.experimental.pallas import tpu as pltpu
from jax.experimental.pallas import tpu_sc as plsc

_N = 10000
_E = 320000
_FIN = 128
_DIM = 95
_G = 64

_NW = 32
_CK = 128
_NCHUNK = 79
_EW = _CK * _NCHUNK
_E2 = _NW * _EW
_NPAD = 10112
_RPS = _NPAD // 16
_FW = _FIN


def _sc_mesh():
    return plsc.VectorSubcoreMesh(core_axis_name="c", subcore_axis_name="s",
                                  num_cores=2, num_subcores=16)


def _sc_gin_body(x_hbm, z_hbm, s_hbm, d_hbm, out_hbm, sidx, didx, rows, acc, sem):
    cid = lax.axis_index("c")
    sid = lax.axis_index("s")
    wid = sid * 2 + cid
    r0 = sid * _RPS
    pltpu.sync_copy(z_hbm.at[pl.ds(r0, _RPS)], acc.at[pl.ds(r0, _RPS)])
    plsc.subcore_barrier()
    e0 = wid * _EW

    def body(i, carry):
        b = e0 + i * _CK
        pltpu.sync_copy(s_hbm.at[pl.ds(b, _CK)], sidx)
        pltpu.sync_copy(d_hbm.at[pl.ds(b, _CK)], didx)
        pltpu.async_copy(x_hbm.at[sidx], rows, sem).wait()
        pltpu.sync_copy(rows, acc.at[didx], add=True)
        return carry

    lax.fori_loop(0, _NCHUNK, body, 0)
    plsc.subcore_barrier()
    pltpu.sync_copy(acc.at[pl.ds(r0, _RPS)],
                    out_hbm.at[pl.ds(cid * _NPAD + r0, _RPS)])


def _sc_gin(xp, zeros_big, srcp, dstp):
    run = pl.kernel(
        _sc_gin_body,
        out_type=jax.ShapeDtypeStruct((2 * _NPAD, _FIN), jnp.float32),
        mesh=_sc_mesh(),
        scratch_types=[
            pltpu.VMEM((_CK,), jnp.int32),
            pltpu.VMEM((_CK,), jnp.int32),
            pltpu.VMEM((_CK, _FIN), jnp.float32),
            pltpu.VMEM_SHARED((_NPAD, _FIN), jnp.float32),
            pltpu.SemaphoreType.DMA,
        ],
    )
    return run(xp, zeros_big, srcp, dstp)


def _sc_gat_body(hp_hbm, z_hbm, s_hbm, d_hbm, as_hbm, ad_hbm, c_hbm,
                 rowp_hbm,
                 sidx, didx, av, asv, adv, cv, rows, acc, sem):
    cid = lax.axis_index("c")
    sid = lax.axis_index("s")
    wid = sid * 2 + cid
    r0 = sid * _RPS
    pltpu.sync_copy(z_hbm.at[pl.ds(r0, _RPS)], acc.at[pl.ds(r0, _RPS)])
    pltpu.sync_copy(as_hbm, asv)
    pltpu.sync_copy(ad_hbm, adv)
    pltpu.sync_copy(c_hbm, cv)
    plsc.subcore_barrier()
    cvec = cv[...]
    e0 = wid * _EW

    def body(i, carry):
        b = e0 + i * _CK
        pltpu.sync_copy(s_hbm.at[pl.ds(b, _CK)], sidx)
        pltpu.sync_copy(d_hbm.at[pl.ds(b, _CK)], didx)
        gather = pltpu.async_copy(hp_hbm.at[sidx], rows, sem)
        for g in range(_CK // 16):
            isv = sidx[pl.ds(g * 16, 16)]
            idv = didx[pl.ds(g * 16, 16)]
            z = plsc.load_gather(asv, [isv]) + plsc.load_gather(adv, [idv])
            e = jnp.where(z >= 0.0, z, z * 0.2)
            ee = jnp.exp(e - cvec)
            av[pl.ds(g * 16, 16)] = ee
        gather.wait()

        def sb(g, c2):
            ev = av[pl.ds(g * 16, 16)]
            for k in range(16):
                a = ev[k]
                for j in range(_FW // 16):
                    sl = pl.ds(j * 16, 16)
                    rows[g * 16 + k, sl] = rows[g * 16 + k, sl] * a
            return c2

        lax.fori_loop(0, _CK // 16, sb, 0)
        pltpu.sync_copy(rows, acc.at[didx], add=True)
        return carry

    lax.fori_loop(0, _NCHUNK, body, 0)
    plsc.subcore_barrier()
    pltpu.sync_copy(acc.at[pl.ds(r0, _RPS)],
                    rowp_hbm.at[pl.ds(cid * _NPAD + r0, _RPS)])


def _sc_gat(hp_p, zeros_wide, srcp, dstp, asp, adp, c16):
    run = pl.kernel(
        _sc_gat_body,
        compiler_params=pltpu.CompilerParams(needs_layout_passes=False),
        out_type=jax.ShapeDtypeStruct((2 * _NPAD, _FW), jnp.float32),
        mesh=_sc_mesh(),
        scratch_types=[
            pltpu.VMEM((_CK,), jnp.int32),
            pltpu.VMEM((_CK,), jnp.int32),
            pltpu.VMEM((_CK,), jnp.float32),
            pltpu.VMEM((_NPAD,), jnp.float32),
            pltpu.VMEM((_NPAD,), jnp.float32),
            pltpu.VMEM((16,), jnp.float32),
            pltpu.VMEM((_CK, _FW), jnp.float32),
            pltpu.VMEM_SHARED((_NPAD, _FW), jnp.float32),
            pltpu.SemaphoreType.DMA,
        ],
    )
    return run(hp_p, zeros_wide, srcp, dstp, asp, adp, c16)


def _tc1_body(x_ref, agg_ref, w1a, b1a, w1b, b1b, gam, bet, gw, attm,
              hp_o, as_o, ad_o, c_o):
    agg = agg_ref[0] + agg_ref[1]
    h = x_ref[...] + agg[:_N]
    h = jnp.maximum(
        jnp.dot(h, w1a[...], preferred_element_type=jnp.float32) + b1a[...], 0.0)
    h = jnp.dot(h, w1b[...], preferred_element_type=jnp.float32) + b1b[...]
    h = jnp.maximum(h, 0.0)
    ones = jnp.full((1, _N), 1.0, jnp.float32)
    mu = jnp.dot(ones, h, preferred_element_type=jnp.float32,
                 precision=lax.Precision.HIGHEST) / _N
    var = jnp.dot(ones, (h - mu) ** 2, preferred_element_type=jnp.float32,
                  precision=lax.Precision.HIGHEST) / _N
    h = (h - mu) / jnp.sqrt(var + 1e-5) * gam[...] + bet[...]
    hp = jnp.dot(h, gw[...], preferred_element_type=jnp.float32)
    asd = jnp.dot(hp, attm[...], preferred_element_type=jnp.float32)
    a_s = asd[:, 0:1]
    a_d = asd[:, 1:2]
    zmax = jnp.max(a_s) + jnp.max(a_d)
    cval = jnp.where(zmax >= 0.0, zmax, zmax * 0.2)
    hp_o[...] = hp
    as_o[...] = a_s
    ad_o[...] = a_d
    c_o[...] = jnp.reshape(cval, (1, 1))


_tc1 = pl.pallas_call(
    _tc1_body,
    out_shape=[
        jax.ShapeDtypeStruct((_N, _FIN), jnp.float32),
        jax.ShapeDtypeStruct((_N, 1), jnp.float32),
        jax.ShapeDtypeStruct((_N, 1), jnp.float32),
        jax.ShapeDtypeStruct((1, 1), jnp.float32),
    ],
)


def _tc2_body(rowp_ref, hp_ref, as_ref, ad_ref, c_ref, batch_ref,
              gatb, fcw, fcb, w10, b10, w20, b20, w11, b11, w21, b21,
              w12, b12, w22, b22, o1_ref, o2_ref, o3_ref):
    z = as_ref[...] + ad_ref[...]
    se = jnp.where(z >= 0.0, z, z * 0.2)
    see = jnp.exp(se - c_ref[...])
    agg = rowp_ref[0] + rowp_ref[1]
    den_e = agg[:_N, _DIM:_DIM + 1]
    inv = 1.0 / (den_e + see)
    out = inv * agg[:_N, :_FIN] + (see * inv) * hp_ref[...] + gatb[...]
    iota = lax.broadcasted_iota(jnp.int32, (128, _N), 0)
    oh = (batch_ref[...] == iota).astype(jnp.float32)
    pooled = jnp.dot(oh, out, preferred_element_type=jnp.float32,
                     precision=lax.Precision.HIGHEST)
    gv = jnp.maximum(
        jnp.dot(pooled, fcw[...], preferred_element_type=jnp.float32) + fcb[...],
        0.0)

    def head(w1, b1, w2, b2):
        t = jnp.maximum(
            jnp.dot(gv, w1[...], preferred_element_type=jnp.float32) + b1[...],
            0.0)
        return jnp.dot(t, w2[...], preferred_element_type=jnp.float32) + b2[...]

    o1 = head(w10, b10, w20, b20)
    o1_ref[...] = 1.0 / (1.0 + jnp.exp(-o1))
    o2_ref[...] = head(w11, b11, w21, b21)
    o3_ref[...] = head(w12, b12, w22, b22)


_tc2 = pl.pallas_call(
    _tc2_body,
    out_shape=[
        jax.ShapeDtypeStruct((128, 128), jnp.float32),
        jax.ShapeDtypeStruct((128, 128), jnp.float32),
        jax.ShapeDtypeStruct((128, 128), jnp.float32),
    ],
)


def kernel(x, edge_index, batch, W1a, b1a, W1b, b1b, bn_gamma, bn_beta, gatW,
           att_src, att_dst, gat_b, fcW, fcb, h0W1, h0b1, h0W2, h0b2,
           h1W1, h1b1, h1W2, h1b2, h2W1, h2b1, h2W2, h2b2):
    f32 = jnp.float32
    src, dst = edge_index[0], edge_index[1]
    pad_e = _E2 - _E
    srcp = jnp.concatenate([src, jnp.full((pad_e,), _N, jnp.int32)])
    dstp = jnp.concatenate([dst, jnp.full((pad_e,), _N, jnp.int32)])
    xp = jnp.concatenate([x, jnp.zeros((_NPAD - _N, _FIN), f32)])
    zeros_big = jnp.zeros((_NPAD, _FIN), f32)

    aggp = _sc_gin(xp, zeros_big, srcp, dstp)

    pad = _FIN - _DIM
    w1a = jnp.pad(W1a, ((0, 0), (0, pad)))
    b1a_p = jnp.pad(b1a, (0, pad)).reshape(1, _FIN)
    w1b = jnp.pad(W1b, ((0, pad), (0, pad)))
    b1b_p = jnp.pad(b1b, (0, pad)).reshape(1, _FIN)
    gam = jnp.pad(bn_gamma, (0, pad)).reshape(1, _FIN)
    bet = jnp.pad(bn_beta, (0, pad)).reshape(1, _FIN)
    gw = jnp.pad(gatW, ((0, pad), (0, pad)))
    attm = jnp.zeros((_FIN, _FIN), f32)
    attm = attm.at[:_DIM, 0].set(att_src).at[:_DIM, 1].set(att_dst)

    hp, a_s, a_d, c = _tc1(x, aggp.reshape(2, _NPAD, _FIN), w1a, b1a_p,
                           w1b, b1b_p, gam, bet, gw, attm)

    hp_w = jnp.concatenate(
        [hp[:, :_DIM], jnp.ones((_N, 1), f32),
         jnp.zeros((_N, _FIN - _DIM - 1), f32)], axis=1)
    hp_p = jnp.concatenate([hp_w, jnp.zeros((_NPAD - _N, _FW), f32)])
    asp = jnp.concatenate([a_s[:, 0], jnp.full((_NPAD - _N,), -1e30, f32)])
    adp = jnp.concatenate([a_d[:, 0], jnp.full((_NPAD - _N,), -1e30, f32)])
    c16 = jnp.broadcast_to(c.reshape(1,), (16,))
    rowp = _sc_gat(hp_p, zeros_big, srcp, dstp, asp, adp, c16)

    batch_row = batch.reshape(1, _N)
    gatb = jnp.pad(gat_b, (0, pad)).reshape(1, _FIN)
    fcw = jnp.pad(fcW, ((0, pad), (0, 256 - 2 * _DIM)))
    fcb_p = jnp.pad(fcb, (0, 256 - 2 * _DIM)).reshape(1, 256)

    def padh(w1, b1, w2, b2):
        return (jnp.pad(w1, ((0, 256 - 2 * _DIM), (0, 128 - 12))),
                jnp.pad(b1, (0, 128 - 12)).reshape(1, 128),
                jnp.pad(w2, ((0, 128 - 12), (0, 127))),
                jnp.pad(b2, (0, 127)).reshape(1, 128))

    w10, b10, w20, b20 = padh(h0W1, h0b1, h0W2, h0b2)
    w11, b11, w21, b21 = padh(h1W1, h1b1, h1W2, h1b2)
    w12, b12, w22, b22 = padh(h2W1, h2b1, h2W2, h2b2)

    o1, o2, o3 = _tc2(rowp.reshape(2, _NPAD, _FW), hp, a_s, a_d, c,
                      batch_row, gatb, fcw, fcb_p, w10, b10, w20, b20,
                      w11, b11, w21, b21, w12, b12, w22, b22)
    return (o1[:_G, :1], o2[:_G, :1], o3[:_G, :1])
